# baseline (speedup 1.0000x reference)
import functools
import jax
import jax.numpy as jnp
from jax import lax
from jax.experimental import pallas as pl
from jax.experimental.pallas import tpu as pltpu
from jax.experimental.pallas import tpu_sc as plsc


def kernel(x, emb):
    b, s, d = x.shape
    NC, NS = 2, 16
    NW = NC * NS
    rows_per_w = s // NW
    CH = 32
    n_chunks = rows_per_w // CH
    mesh = plsc.VectorSubcoreMesh(core_axis_name="c", subcore_axis_name="s")

    @functools.partial(
        pl.kernel,
        mesh=mesh,
        out_type=jax.ShapeDtypeStruct((b, s, d), jnp.float32),
        scratch_types=[
            pltpu.VMEM((2, CH, d), jnp.float32),
            pltpu.SemaphoreType.DMA,
            pltpu.SemaphoreType.DMA,
            pltpu.SemaphoreType.DMA,
        ],
    )
    def sc_copy(emb_hbm, out_hbm, buf, rsem, wsem0, wsem1):
        wid = lax.axis_index("s") * NC + lax.axis_index("c")
        base = wid * rows_per_w
        wsems = (wsem0, wsem1)

        def read(i):
            return pltpu.make_async_copy(
                emb_hbm.at[pl.ds(base + i * CH, CH)], buf.at[i % 2], rsem)

        def writes(i):
            return [
                pltpu.make_async_copy(
                    buf.at[i % 2],
                    out_hbm.at[bi].at[pl.ds(base + i * CH, CH)],
                    wsems[i % 2])
                for bi in range(b)
            ]

        for i in range(n_chunks):
            r = read(i)
            r.start()
            r.wait()
            ws = writes(i)
            for w in ws:
                w.start()
            for w in ws:
                w.wait()

    return sc_copy(emb[:s])


# speedup vs baseline: 1.0101x; 1.0101x over previous
import functools
import jax
import jax.numpy as jnp
from jax import lax
from jax.experimental import pallas as pl
from jax.experimental.pallas import tpu as pltpu
from jax.experimental.pallas import tpu_sc as plsc


def kernel(x, emb):
    b, s, d = x.shape
    NC, NS = 2, 16
    NW = NC * NS
    rows_per_w = s // NW
    CH = 32
    n_chunks = rows_per_w // CH
    mesh = plsc.VectorSubcoreMesh(core_axis_name="c", subcore_axis_name="s")

    @functools.partial(
        pl.kernel,
        mesh=mesh,
        out_type=jax.ShapeDtypeStruct((b, s, d), jnp.float32),
        scratch_types=[
            pltpu.VMEM((2, CH, d), jnp.float32),
            pltpu.SemaphoreType.DMA,
            pltpu.SemaphoreType.DMA,
            pltpu.SemaphoreType.DMA,
        ],
    )
    def sc_copy(emb_hbm, out_hbm, buf, rsem, wsem0, wsem1):
        wid = lax.axis_index("s") * NC + lax.axis_index("c")
        base = wid * rows_per_w
        wsems = (wsem0, wsem1)

        def read(i):
            return pltpu.make_async_copy(
                emb_hbm.at[pl.ds(base + i * CH, CH)], buf.at[i % 2], rsem)

        def writes(i):
            return [
                pltpu.make_async_copy(
                    buf.at[i % 2],
                    out_hbm.at[bi].at[pl.ds(base + i * CH, CH)],
                    wsems[i % 2])
                for bi in range(b)
            ]

        read(0).start()
        for i in range(n_chunks):
            read(i).wait()
            for w in writes(i):
                w.start()
            if i + 1 < n_chunks:
                if i >= 1:
                    for w in writes(i - 1):
                        w.wait()
                read(i + 1).start()
        for w in writes(n_chunks - 2):
            w.wait()
        for w in writes(n_chunks - 1):
            w.wait()

    return sc_copy(emb[:s])


# speedup vs baseline: 1.0520x; 1.0415x over previous
import functools
import jax
import jax.numpy as jnp
from jax import lax
from jax.experimental import pallas as pl
from jax.experimental.pallas import tpu as pltpu
from jax.experimental.pallas import tpu_sc as plsc


def kernel(x, emb):
    b, s, d = x.shape
    NC, NS = 2, 16
    NW = NC * NS
    rows_per_w = s // NW
    CH = 64
    n_chunks = rows_per_w // CH
    mesh = plsc.VectorSubcoreMesh(core_axis_name="c", subcore_axis_name="s")

    @functools.partial(
        pl.kernel,
        mesh=mesh,
        out_type=jax.ShapeDtypeStruct((b, s, d), jnp.float32),
        scratch_types=[
            pltpu.VMEM((CH, d), jnp.float32),
            pltpu.SemaphoreType.DMA,
            pltpu.SemaphoreType.DMA,
        ],
    )
    def sc_copy(emb_hbm, out_hbm, buf, rsem, wsem):
        wid = lax.axis_index("s") * NC + lax.axis_index("c")
        base = wid * rows_per_w

        for i in range(n_chunks):
            r = pltpu.make_async_copy(
                emb_hbm.at[pl.ds(base + i * CH, CH)], buf, rsem)
            r.start()
            r.wait()
            ws = [
                pltpu.make_async_copy(
                    buf, out_hbm.at[bi].at[pl.ds(base + i * CH, CH)], wsem)
                for bi in range(b)
            ]
            for w in ws:
                w.start()
            for w in ws:
                w.wait()

    return sc_copy(emb[:s])


# speedup vs baseline: 1.3442x; 1.2778x over previous
import functools
import jax
import jax.numpy as jnp
from jax import lax
from jax.experimental import pallas as pl
from jax.experimental.pallas import tpu as pltpu
from jax.experimental.pallas import tpu_sc as plsc


def kernel(x, emb):
    b, s, d = x.shape
    NC, NS = 2, 16
    NW = NC * NS
    rows_per_w = s // NW
    CH = 64
    n_chunks = rows_per_w // CH
    mesh = plsc.VectorSubcoreMesh(core_axis_name="c", subcore_axis_name="s")

    @functools.partial(
        pl.kernel,
        mesh=mesh,
        out_type=jax.ShapeDtypeStruct((b, s, d), jnp.float32),
        scratch_types=[
            pltpu.VMEM((CH, d), jnp.float32),
            pltpu.SemaphoreType.DMA,
            pltpu.SemaphoreType.DMA,
        ],
    )
    def sc_copy(emb_hbm, out_hbm, buf, rsem, wsem):
        wid = lax.axis_index("s") * NC + lax.axis_index("c")
        base = wid * rows_per_w

        for i in range(n_chunks):
            r = pltpu.make_async_copy(
                emb_hbm.at[pl.ds(base + i * CH, CH)], buf, rsem)
            r.start()
            r.wait()
            ws = [
                pltpu.make_async_copy(
                    buf, out_hbm.at[bi].at[pl.ds(base + i * CH, CH)], wsem)
                for bi in range(b)
            ]
            for w in ws:
                w.start()
            for w in ws:
                w.wait()

    return sc_copy(emb)
